# baseline (speedup 1.0000x reference)
import functools

import jax
import jax.numpy as jnp
from jax import lax
from jax.experimental import pallas as pl
from jax.experimental.pallas import tpu as pltpu
from jax.experimental.pallas import tpu_sc as plsc

_NE = 64
_TK = 8
_TB = 1024
_NW = 32
_HC = 256


def _logits_block(x_ref, w_ref, b_ref, o_ref):
    o_ref[...] = (
        jax.lax.dot_general(
            x_ref[...],
            w_ref[...],
            (((1,), (1,)), ((), ())),
            preferred_element_type=jnp.float32,
        )
        + b_ref[...]
    )


def _tc_logits(x, W, b):
    n_tok, E = x.shape
    return pl.pallas_call(
        _logits_block,
        grid=(n_tok // _TB,),
        in_specs=[
            pl.BlockSpec((_TB, E), lambda i: (i, 0)),
            pl.BlockSpec((_NE, E), lambda i: (0, 0)),
            pl.BlockSpec((1, _NE), lambda i: (0, 0)),
        ],
        out_specs=pl.BlockSpec((_TB, _NE), lambda i: (i, 0)),
        out_shape=jax.ShapeDtypeStruct((n_tok, _NE), jnp.float32),
    )(x, W, b.reshape(1, _NE))


def _route_body(lg_hbm, out_hbm, idx_hbm, lg_v, out_v, idx_v):
    wid = lax.axis_index("s") * 2 + lax.axis_index("c")
    ch = _HC * 2
    iota = lax.iota(jnp.int32, 16)
    zero16 = jnp.zeros((16,), jnp.float32)
    neg_inf = jnp.float32(-jnp.inf)

    for h in range(2):
        base = wid * ch + h * _HC
        pltpu.sync_copy(lg_hbm.at[pl.ds(base * _NE, _HC * _NE)], lg_v)

        def _zero(i, _):
            out_v[pl.ds(i * 16, 16)] = zero16
            return 0

        lax.fori_loop(0, _HC * _NE // 16, _zero, 0)

        def _group(g, _):
            t0 = g * 16
            rowbase = (t0 + iota) * _NE
            b_v = [jnp.full((16,), neg_inf, jnp.float32) for _ in range(_TK)]
            b_i = [jnp.zeros((16,), jnp.int32) for _ in range(_TK)]
            for e in range(_NE):
                cur_v = plsc.load_gather(lg_v, [rowbase + e])
                cur_i = jnp.full((16,), e, jnp.int32)
                for k in range(_TK):
                    swap = cur_v > b_v[k]
                    nv = jnp.where(swap, cur_v, b_v[k])
                    cur_v = jnp.where(swap, b_v[k], cur_v)
                    b_v[k] = nv
                    ni = jnp.where(swap, cur_i, b_i[k])
                    cur_i = jnp.where(swap, b_i[k], cur_i)
                    b_i[k] = ni
            m = b_v[0]
            probs = [jnp.exp(b_v[k] - m) for k in range(_TK)]
            denom = probs[0]
            for k in range(1, _TK):
                denom = denom + probs[k]
            inv = 1.0 / denom
            idxbase = (t0 + iota) * _TK
            for k in range(_TK):
                plsc.store_scatter(out_v, [rowbase + b_i[k]], probs[k] * inv)
                plsc.store_scatter(idx_v, [idxbase + k], b_i[k])
            return 0

        lax.fori_loop(0, _HC // 16, _group, 0)
        pltpu.sync_copy(out_v, out_hbm.at[pl.ds(base * _NE, _HC * _NE)])
        pltpu.sync_copy(idx_v, idx_hbm.at[pl.ds(base * _TK, _HC * _TK)])


def _sc_route(logits_flat, n_tok):
    mesh = plsc.VectorSubcoreMesh(core_axis_name="c", subcore_axis_name="s")
    k = functools.partial(
        pl.kernel,
        out_type=[
            jax.ShapeDtypeStruct((n_tok * _NE,), jnp.float32),
            jax.ShapeDtypeStruct((n_tok * _TK,), jnp.int32),
        ],
        mesh=mesh,
        compiler_params=pltpu.CompilerParams(needs_layout_passes=False),
        scratch_types=[
            pltpu.VMEM((_HC * _NE,), jnp.float32),
            pltpu.VMEM((_HC * _NE,), jnp.float32),
            pltpu.VMEM((_HC * _TK,), jnp.int32),
        ],
    )(_route_body)
    return k(logits_flat)


@jax.jit
def kernel(mh_output, W, b):
    B, S, E = mh_output.shape
    n_tok = B * S
    x = mh_output.reshape(n_tok, E)
    logits = _tc_logits(x, W, b)
    router_flat, idx_flat = _sc_route(logits.reshape(n_tok * _NE), n_tok)
    return (
        router_flat.reshape(B, S, _NE),
        idx_flat.reshape(B, S, _TK),
    )


# speedup vs baseline: 1.9469x; 1.9469x over previous
import jax
import jax.numpy as jnp
from jax.experimental import pallas as pl

_NUM_EXPERTS = 64
_TOP_K = 8
_TB = 1024
_SUB = 2


def _route_tile(logits):
    tb = logits.shape[1]
    fiota = jax.lax.broadcasted_iota(jnp.int32, (_NUM_EXPERTS, tb), 0).astype(
        jnp.float32
    )
    work = logits
    idx_rows = []
    top_val = None
    neg_inf = jnp.float32(-jnp.inf)
    for k in range(_TOP_K):
        m = jnp.max(work, axis=0, keepdims=True)
        if k == 0:
            top_val = m
        idx = jnp.min(
            jnp.where(work == m, fiota, jnp.float32(_NUM_EXPERTS)),
            axis=0,
            keepdims=True,
        )
        work = jnp.where(fiota == idx, neg_inf, work)
        idx_rows.append(idx)

    e = jnp.where(work == neg_inf, jnp.exp(logits - top_val), 0.0)
    denom = jnp.sum(e, axis=0, keepdims=True)
    idxs = jnp.concatenate(idx_rows, axis=0)
    return (e / denom).T, idxs.astype(jnp.int32)


def _router_block(x_ref, w_ref, b_ref, out_ref, idx_ref):
    w = w_ref[...]
    st = _TB // _SUB
    for s in range(_SUB):
        logits = jax.lax.dot_general(
            w,
            x_ref[pl.ds(s * st, st), :],
            (((1,), (1,)), ((), ())),
            preferred_element_type=jnp.float32,
        )
        logits = logits + b_ref[...]
        probs, idxs = _route_tile(logits)
        out_ref[pl.ds(s * st, st), :] = probs
        idx_ref[:, pl.ds(s * st, st)] = idxs


@jax.jit
def kernel(mh_output, W, b):
    B, S, E = mh_output.shape
    n_tok = B * S
    x = mh_output.reshape(n_tok, E)
    grid = (n_tok // _TB,)
    router, idx = pl.pallas_call(
        _router_block,
        grid=grid,
        in_specs=[
            pl.BlockSpec((_TB, E), lambda i: (i, 0)),
            pl.BlockSpec((_NUM_EXPERTS, E), lambda i: (0, 0)),
            pl.BlockSpec((_NUM_EXPERTS, 1), lambda i: (0, 0)),
        ],
        out_specs=[
            pl.BlockSpec((_TB, _NUM_EXPERTS), lambda i: (i, 0)),
            pl.BlockSpec((_TOP_K, _TB), lambda i: (0, i)),
        ],
        out_shape=[
            jax.ShapeDtypeStruct((n_tok, _NUM_EXPERTS), jnp.float32),
            jax.ShapeDtypeStruct((_TOP_K, n_tok), jnp.int32),
        ],
    )(x, W, b.reshape(_NUM_EXPERTS, 1))
    return router.reshape(B, S, _NUM_EXPERTS), idx.T.reshape(B, S, _TOP_K)


# speedup vs baseline: 2.0223x; 1.0387x over previous
import jax
import jax.numpy as jnp
from jax.experimental import pallas as pl

_NUM_EXPERTS = 64
_TOP_K = 8
_TB = 1024
_SUB = 2


def _route_tile(logits):
    tb = logits.shape[1]
    fiota = jax.lax.broadcasted_iota(jnp.int32, (_NUM_EXPERTS, tb), 0).astype(
        jnp.float32
    )
    work = logits
    idx_rows = []
    top_val = None
    neg_inf = jnp.float32(-jnp.inf)
    for k in range(_TOP_K):
        m = jnp.max(work, axis=0, keepdims=True)
        if k == 0:
            top_val = m
        idx = jnp.min(
            jnp.where(work == m, fiota, jnp.float32(_NUM_EXPERTS)),
            axis=0,
            keepdims=True,
        )
        work = jnp.where(fiota == idx, neg_inf, work)
        idx_rows.append(idx)

    e = jnp.where(work == neg_inf, jnp.exp(logits - top_val), 0.0)
    denom = jnp.sum(e, axis=0, keepdims=True)
    idxs = jnp.concatenate(idx_rows, axis=0)
    return e / denom, idxs.astype(jnp.int32)


def _router_block(x_ref, w_ref, b_ref, out_ref, idx_ref):
    w = w_ref[...]
    st = _TB // _SUB
    for s in range(_SUB):
        logits = jax.lax.dot_general(
            w,
            x_ref[pl.ds(s * st, st), :],
            (((1,), (1,)), ((), ())),
            preferred_element_type=jnp.float32,
        )
        logits = logits + b_ref[...]
        probs, idxs = _route_tile(logits)
        out_ref[:, pl.ds(s * st, st)] = probs
        idx_ref[:, pl.ds(s * st, st)] = idxs


@jax.jit
def kernel(mh_output, W, b):
    B, S, E = mh_output.shape
    n_tok = B * S
    x = mh_output.reshape(n_tok, E)
    grid = (n_tok // _TB,)
    router, idx = pl.pallas_call(
        _router_block,
        grid=grid,
        in_specs=[
            pl.BlockSpec((_TB, E), lambda i: (i, 0)),
            pl.BlockSpec((_NUM_EXPERTS, E), lambda i: (0, 0)),
            pl.BlockSpec((_NUM_EXPERTS, 1), lambda i: (0, 0)),
        ],
        out_specs=[
            pl.BlockSpec((_NUM_EXPERTS, _TB), lambda i: (0, i)),
            pl.BlockSpec((_TOP_K, _TB), lambda i: (0, i)),
        ],
        out_shape=[
            jax.ShapeDtypeStruct((_NUM_EXPERTS, n_tok), jnp.float32),
            jax.ShapeDtypeStruct((_TOP_K, n_tok), jnp.int32),
        ],
    )(x, W, b.reshape(_NUM_EXPERTS, 1))
    return router.T.reshape(B, S, _NUM_EXPERTS), idx.T.reshape(B, S, _TOP_K)
